# Initial kernel scaffold; baseline (speedup 1.0000x reference)
import jax
import jax.numpy as jnp
from jax.experimental import pallas as pl


def kernel(W, R, feat_idx):
    raise NotImplementedError("write your pallas kernel here")


# speedup vs baseline: 231.8184x; 231.8184x over previous
import functools

import jax
import jax.numpy as jnp
from jax import lax
from jax.experimental import pallas as pl
from jax.experimental.pallas import tpu as pltpu
from jax.experimental.pallas import tpu_sc as plsc

_W_ADJ = 0.4
_INV_R_ADJ = 0.1
_VOCAB1 = 100001
_VPAD = 100352
_B = 16384
_L = 200
_NCLS = 2
_NSUB = 16
_ROWS_PER_W = _B // _NSUB
_CHUNK = 32
_NCHUNK = _ROWS_PER_W // _CHUNK


def _fuse_body(w_ref, r_ref, t_ref):
    t_ref[...] = (w_ref[...] + _W_ADJ) * r_ref[...] * _INV_R_ADJ


def _fuse_tables(wp, rt):
    bk = _VPAD // 8
    return pl.pallas_call(
        _fuse_body,
        grid=(_VPAD // bk,),
        in_specs=[
            pl.BlockSpec((1, bk), lambda j: (0, j)),
            pl.BlockSpec((_NCLS, bk), lambda j: (0, j)),
        ],
        out_specs=pl.BlockSpec((_NCLS, bk), lambda j: (0, j)),
        out_shape=jax.ShapeDtypeStruct((_NCLS, _VPAD), jnp.float32),
    )(wp, rt)


def _sc_body(t_hbm, idx_hbm, out_hbm, table_v, idx_v, out_v):
    cls = lax.axis_index("c")
    w2 = lax.axis_index("s")
    row0 = w2 * _ROWS_PER_W

    pltpu.sync_copy(t_hbm.at[cls], table_v)

    lane = lax.iota(jnp.int32, 16)

    def chunk_body(chunk, _):
        pltpu.sync_copy(
            idx_hbm.at[pl.ds((row0 + chunk * _CHUNK) * _L, _CHUNK * _L)], idx_v)

        for sub in range(_CHUNK // 16):
            base = (lane + (sub * 16)) * _L

            def tok_body(l, acc):
                tok = plsc.load_gather(idx_v, [base + l])
                return acc + plsc.load_gather(table_v, [tok])

            acc = lax.fori_loop(0, _L, tok_body,
                                jnp.zeros((16,), jnp.float32))
            out_v[pl.ds(chunk * _CHUNK + sub * 16, 16)] = acc
        return ()

    lax.fori_loop(0, _NCHUNK, chunk_body, ())

    pltpu.sync_copy(out_v, out_hbm.at[cls, pl.ds(row0, _ROWS_PER_W)])


@functools.partial(
    pl.kernel,
    mesh=plsc.VectorSubcoreMesh(core_axis_name="c", subcore_axis_name="s"),
    out_type=jax.ShapeDtypeStruct((_NCLS, _B), jnp.float32),
    compiler_params=pltpu.CompilerParams(
        needs_layout_passes=False, use_tc_tiling_on_sc=False),
    scratch_types=[
        pltpu.VMEM((_VPAD,), jnp.float32),
        pltpu.VMEM((_CHUNK * _L,), jnp.int32),
        pltpu.VMEM((_ROWS_PER_W,), jnp.float32),
    ],
)
def _sc_gather_sum(t_hbm, idx_hbm, out_hbm, table_v, idx_v, out_v):
    _sc_body(t_hbm, idx_hbm, out_hbm, table_v, idx_v, out_v)


def kernel(W, R, feat_idx):
    wp = jnp.pad(W[:, 0], (0, _VPAD - _VOCAB1)).reshape(1, _VPAD)
    rt = jnp.pad(R, ((0, _VPAD - _VOCAB1), (0, 0))).T
    fused = _fuse_tables(wp, rt)
    out2 = _sc_gather_sum(fused, feat_idx.astype(jnp.int32).reshape(-1))
    return out2.T


# speedup vs baseline: 448.7300x; 1.9357x over previous
import functools

import jax
import jax.numpy as jnp
from jax import lax
from jax.experimental import pallas as pl
from jax.experimental.pallas import tpu as pltpu
from jax.experimental.pallas import tpu_sc as plsc

_W_ADJ = 0.4
_INV_R_ADJ = 0.1
_VOCAB1 = 100001
_VPAD = 100352
_B = 16384
_L = 200
_NCLS = 2
_NSUB = 16
_ROWS_PER_W = _B // _NSUB
_CHUNK = 32
_NCHUNK = _ROWS_PER_W // _CHUNK


def _fuse_body(w_ref, r_ref, t_ref):
    t_ref[...] = (w_ref[...] + _W_ADJ) * r_ref[...] * _INV_R_ADJ


def _fuse_tables(wp, rt):
    bk = _VPAD // 8
    return pl.pallas_call(
        _fuse_body,
        grid=(_VPAD // bk,),
        in_specs=[
            pl.BlockSpec((1, bk), lambda j: (0, j)),
            pl.BlockSpec((_NCLS, bk), lambda j: (0, j)),
        ],
        out_specs=pl.BlockSpec((_NCLS, bk), lambda j: (0, j)),
        out_shape=jax.ShapeDtypeStruct((_NCLS, _VPAD), jnp.float32),
    )(wp, rt)


_UNROLL = 8


def _sc_body(t_hbm, idx_hbm, out_hbm, table_v, idx_a, idx_b, out_v, sem_a,
             sem_b):
    cls = lax.axis_index("c")
    w2 = lax.axis_index("s")
    row0 = w2 * _ROWS_PER_W

    pltpu.sync_copy(t_hbm.at[cls], table_v)

    lane = lax.iota(jnp.int32, 16)

    def chunk_src(chunk):
        return idx_hbm.at[pl.ds((row0 + chunk * _CHUNK) * _L, _CHUNK * _L)]

    def consume(chunk, buf):
        for sub in range(_CHUNK // 16):
            base = (lane + (sub * 16)) * _L

            def tok_body(i, acc):
                for k in range(_UNROLL):
                    tok = plsc.load_gather(buf, [base + (i * _UNROLL + k)])
                    acc = acc + plsc.load_gather(table_v, [tok])
                return acc

            acc = lax.fori_loop(0, _L // _UNROLL, tok_body,
                                jnp.zeros((16,), jnp.float32))
            out_v[pl.ds(chunk * _CHUNK + sub * 16, 16)] = acc

    pltpu.async_copy(chunk_src(0), idx_a, sem_a)

    def pair_body(g, _):
        c0 = g * 2
        pltpu.async_copy(chunk_src(c0 + 1), idx_b, sem_b)
        pltpu.make_async_copy(chunk_src(c0), idx_a, sem_a).wait()
        consume(c0, idx_a)

        @pl.when(g < _NCHUNK // 2 - 1)
        def _():
            pltpu.async_copy(chunk_src(c0 + 2), idx_a, sem_a)

        pltpu.make_async_copy(chunk_src(c0 + 1), idx_b, sem_b).wait()
        consume(c0 + 1, idx_b)
        return ()

    lax.fori_loop(0, _NCHUNK // 2, pair_body, ())

    pltpu.sync_copy(out_v, out_hbm.at[cls, pl.ds(row0, _ROWS_PER_W)])


@functools.partial(
    pl.kernel,
    mesh=plsc.VectorSubcoreMesh(core_axis_name="c", subcore_axis_name="s"),
    out_type=jax.ShapeDtypeStruct((_NCLS, _B), jnp.float32),
    compiler_params=pltpu.CompilerParams(
        needs_layout_passes=False, use_tc_tiling_on_sc=False),
    scratch_types=[
        pltpu.VMEM((_VPAD,), jnp.float32),
        pltpu.VMEM((_CHUNK * _L,), jnp.int32),
        pltpu.VMEM((_CHUNK * _L,), jnp.int32),
        pltpu.VMEM((_ROWS_PER_W,), jnp.float32),
        pltpu.SemaphoreType.DMA,
        pltpu.SemaphoreType.DMA,
    ],
)
def _sc_gather_sum(t_hbm, idx_hbm, out_hbm, table_v, idx_a, idx_b, out_v,
                   sem_a, sem_b):
    _sc_body(t_hbm, idx_hbm, out_hbm, table_v, idx_a, idx_b, out_v, sem_a,
             sem_b)


def kernel(W, R, feat_idx):
    wp = jnp.pad(W[:, 0], (0, _VPAD - _VOCAB1)).reshape(1, _VPAD)
    rt = jnp.pad(R, ((0, _VPAD - _VOCAB1), (0, 0))).T
    fused = _fuse_tables(wp, rt)
    out2 = _sc_gather_sum(fused, feat_idx.astype(jnp.int32).reshape(-1))
    return out2.T
